# baseline (speedup 1.0000x reference)
import functools

import jax
import jax.numpy as jnp
from jax import lax
from jax.experimental import pallas as pl
from jax.experimental.pallas import tpu as pltpu
from jax.experimental.pallas import tpu_sc as plsc

N, H, E, D = 10000, 5000, 320000, 128
NC, NS = 2, 16
NW = NC * NS
C = 80
NCH = 126
SBCH = 42
NSB = NCH // SBCH
EPT = NCH * C
E_PAD = EPT * NW
N_PAD = 10240
H_PAD = 5120
NPT = N_PAD // NS
HPT = H_PAD // NS
NV = D // 16

_MESH = plsc.VectorSubcoreMesh(
    core_axis_name="c", subcore_axis_name="s", num_cores=NC, num_subcores=NS
)

_F32 = jnp.float32
_I32 = jnp.int32


def _row_op(n, fn):
    def body(i, carry):
        for r in range(NV):
            fn(i, pl.ds(r * 16, 16))
        return carry

    lax.fori_loop(0, n, body, 0)


def _scale(buf, valsv, jj):
    def group(g, carry2):
        vv = valsv[jj, pl.ds(g * 16, 16)]
        for e in range(16):
            i = g * 16 + e
            v = vv[e]
            for r in range(NV):
                sl = pl.ds(r * 16, 16)
                buf[i, sl] = buf[i, sl] * v
        return carry2

    lax.fori_loop(0, C // 16, group, 0)


def _edge_pass(table_hbm, acc, wid, cols_hbm, rows_hbm, vals_hbm,
               colsv, rowsv, valsv, bufs, gsems, ssems):
    def chunk_phase(jj, b, prefetch, wait_ss):
        nb = (b + 2) % 3
        pltpu.make_async_copy(table_hbm.at[colsv.at[jj]], bufs[b],
                              gsems[b]).wait()
        _scale(bufs[b], valsv, jj)
        pltpu.async_copy(bufs[b], acc.at[rowsv.at[jj]], ssems[b], add=True)
        if prefetch:
            if wait_ss:
                pltpu.make_async_copy(bufs[nb], acc.at[rowsv.at[jj]],
                                      ssems[nb]).wait()
            pltpu.async_copy(table_hbm.at[colsv.at[jj + 2]], bufs[nb],
                             gsems[nb])

    def sb_body(sb, carry):
        pltpu.sync_copy(cols_hbm.at[wid, sb], colsv)
        pltpu.sync_copy(rows_hbm.at[wid, sb], rowsv)
        pltpu.sync_copy(vals_hbm.at[wid, sb], valsv)
        pltpu.async_copy(table_hbm.at[colsv.at[0]], bufs[0], gsems[0])
        pltpu.async_copy(table_hbm.at[colsv.at[1]], bufs[1], gsems[1])
        chunk_phase(0, 0, prefetch=True, wait_ss=False)
        chunk_phase(1, 1, prefetch=True, wait_ss=True)
        chunk_phase(2, 2, prefetch=True, wait_ss=True)

        def triple(t, carry2):
            jj = 3 * t
            chunk_phase(jj, 0, prefetch=True, wait_ss=True)
            chunk_phase(jj + 1, 1, prefetch=True, wait_ss=True)
            chunk_phase(jj + 2, 2, prefetch=True, wait_ss=True)
            return carry2

        lax.fori_loop(1, SBCH // 3 - 1, triple, 0)
        chunk_phase(SBCH - 3, 0, prefetch=True, wait_ss=True)
        chunk_phase(SBCH - 2, 1, prefetch=False, wait_ss=False)
        chunk_phase(SBCH - 1, 2, prefetch=False, wait_ss=False)
        for b in range(3):
            pltpu.make_async_copy(bufs[b], acc.at[rowsv.at[SBCH - 1]],
                                  ssems[b]).wait()
        return carry

    lax.fori_loop(0, NSB, sb_body, 0)


_SPMM_SCRATCH = [
    pltpu.VMEM((SBCH, C), _I32),
    pltpu.VMEM((SBCH, C), _I32),
    pltpu.VMEM((SBCH, C), _F32),
    pltpu.VMEM((C, D), _F32),
    pltpu.VMEM((C, D), _F32),
    pltpu.VMEM((C, D), _F32),
    pltpu.SemaphoreType.DMA,
    pltpu.SemaphoreType.DMA,
    pltpu.SemaphoreType.DMA,
    pltpu.SemaphoreType.DMA,
    pltpu.SemaphoreType.DMA,
    pltpu.SemaphoreType.DMA,
]


def _spmm_core(table_hbm, acc_rows, c, s, zeros_hbm, out_hbm, acc,
               cols_hbm, rows_hbm, vals_hbm,
               colsv, rowsv, valsv, bufa, bufb, bufc,
               gsa, gsb, gsc, ssa, ssb, ssc):
    wid = c * NS + s
    rpt = acc_rows // NS
    pltpu.sync_copy(zeros_hbm.at[pl.ds(s * rpt, rpt)],
                    acc.at[pl.ds(s * rpt, rpt)])
    plsc.subcore_barrier()
    _edge_pass(table_hbm, acc, wid, cols_hbm, rows_hbm, vals_hbm,
               colsv, rowsv, valsv, (bufa, bufb, bufc),
               (gsa, gsb, gsc), (ssa, ssb, ssc))
    plsc.subcore_barrier()
    pltpu.sync_copy(acc.at[pl.ds(s * rpt, rpt)],
                    out_hbm.at[c].at[pl.ds(s * rpt, rpt)])


@functools.partial(
    pl.kernel,
    out_type=jax.ShapeDtypeStruct((NC, H_PAD, D), _F32),
    mesh=_MESH,
    scratch_types=(
        [pltpu.MemorySpace.VMEM_SHARED((H_PAD, D), _F32)] + _SPMM_SCRATCH),
)
def _spmm_tar1(x_hbm, cols_hbm, rows_hbm, vals_hbm, zeros_hbm, out_hbm,
               acc, *rest):
    c = lax.axis_index("c")
    s = lax.axis_index("s")
    _spmm_core(x_hbm, H_PAD, c, s, zeros_hbm, out_hbm, acc,
               cols_hbm, rows_hbm, vals_hbm, *rest)


@functools.partial(
    pl.kernel,
    out_type=(
        jax.ShapeDtypeStruct((NC, N_PAD, D), _F32),
        jax.ShapeDtypeStruct((NC, H_PAD, D), _F32),
    ),
    mesh=_MESH,
    scratch_types=(
        [pltpu.MemorySpace.VMEM_SHARED((N_PAD, D), _F32)] + _SPMM_SCRATCH),
)
def _spmm_src(p_hbm, cols_hbm, rows_hbm, vals_hbm, zeros_hbm,
              out_hbm, mt_hbm, acc, colsv, rowsv, valsv,
              bufa, bufb, bufc, *sems):
    c = lax.axis_index("c")
    s = lax.axis_index("s")
    for k in range(HPT // C):
        off = s * HPT + k * C
        pltpu.sync_copy(p_hbm.at[0].at[pl.ds(off, C)], bufa)
        pltpu.sync_copy(p_hbm.at[1].at[pl.ds(off, C)], bufb)

        def combine(i, sl):
            bufa[i, sl] = bufa[i, sl] + bufb[i, sl]

        _row_op(C, combine)
        pltpu.sync_copy(bufa, mt_hbm.at[c].at[pl.ds(off, C)])
    _spmm_core(mt_hbm.at[c], N_PAD, c, s, zeros_hbm, out_hbm, acc,
               cols_hbm, rows_hbm, vals_hbm,
               colsv, rowsv, valsv, bufa, bufb, bufc, *sems)


@functools.partial(
    pl.kernel,
    out_type=(
        jax.ShapeDtypeStruct((NC, H_PAD, D), _F32),
        jax.ShapeDtypeStruct((NC, N_PAD, D), _F32),
        jax.ShapeDtypeStruct((NC, N_PAD, D), _F32),
    ),
    mesh=_MESH,
    scratch_types=(
        [pltpu.MemorySpace.VMEM_SHARED((H_PAD, D), _F32)] + _SPMM_SCRATCH),
)
def _spmm_tar2(x_hbm, q_hbm, cols_hbm, rows_hbm, vals_hbm, zeros_hbm,
               out_hbm, x1_hbm, r1_hbm, acc, colsv, rowsv, valsv,
               bufa, bufb, bufc, *sems):
    c = lax.axis_index("c")
    s = lax.axis_index("s")
    for k in range(NPT // C):
        off = s * NPT + k * C
        pltpu.sync_copy(x_hbm.at[pl.ds(off, C)], bufa)
        pltpu.sync_copy(q_hbm.at[0].at[pl.ds(off, C)], bufb)
        pltpu.sync_copy(q_hbm.at[1].at[pl.ds(off, C)], bufc)

        def stage(i, sl):
            r1 = jnp.maximum(bufb[i, sl] + bufc[i, sl], 0.0)
            bufb[i, sl] = r1
            bufc[i, sl] = bufa[i, sl] + r1

        _row_op(C, stage)
        pltpu.sync_copy(bufc, x1_hbm.at[c].at[pl.ds(off, C)])
        pltpu.sync_copy(bufb, r1_hbm.at[c].at[pl.ds(off, C)])
    _spmm_core(x1_hbm.at[c], H_PAD, c, s, zeros_hbm, out_hbm, acc,
               cols_hbm, rows_hbm, vals_hbm,
               colsv, rowsv, valsv, bufa, bufb, bufc, *sems)


KC = 80
KNCH = N // KC


@functools.partial(
    pl.kernel,
    out_type=jax.ShapeDtypeStruct((N, D), _F32),
    mesh=_MESH,
    scratch_types=[
        pltpu.VMEM((KC, D), _F32),
        pltpu.VMEM((KC, D), _F32),
        pltpu.VMEM((KC, D), _F32),
        pltpu.VMEM((KC, D), _F32),
        pltpu.VMEM((16,), _F32),
    ],
)
def _final(x0_hbm, r1_hbm, q_hbm, la_hbm, out_hbm, bx, b1, b2, b3, law):
    c = lax.axis_index("c")
    s = lax.axis_index("s")
    wid = c * NS + s
    pltpu.sync_copy(la_hbm, law)
    wv = law[...]
    ev = jnp.exp(wv - wv[0])
    ssum = ev[0] + ev[1] + ev[2]
    bits = lax.bitcast_convert_type(ssum, _I32)
    r = lax.bitcast_convert_type(jnp.int32(0x7EF127EA) - bits, _F32)
    for _ in range(5):
        r = r * (2.0 - ssum * r)
    w1 = ev[1] * r
    w2 = ev[2] * r
    a = w1 + w2
    for k0 in range((KNCH + NW - 1) // NW):
        j = wid + k0 * NW

        @pl.when(j < KNCH)
        def _():
            off = j * KC
            pltpu.sync_copy(x0_hbm.at[pl.ds(off, KC)], bx)
            pltpu.sync_copy(r1_hbm.at[c].at[pl.ds(off, KC)], b1)
            pltpu.sync_copy(q_hbm.at[0].at[pl.ds(off, KC)], b2)
            pltpu.sync_copy(q_hbm.at[1].at[pl.ds(off, KC)], b3)

            def mix(i, sl):
                r2 = jnp.maximum(b2[i, sl] + b3[i, sl], 0.0)
                bx[i, sl] = bx[i, sl] + a * b1[i, sl] + w2 * r2

            _row_op(KC, mix)
            pltpu.sync_copy(bx, out_hbm.at[pl.ds(off, KC)])


def _pad_edges(rows, cols, vals, nrows, ncols):
    pad = E_PAD - E
    ar = jnp.arange(pad, dtype=_I32)
    rows = jnp.concatenate([rows.astype(_I32), ar % nrows])
    cols = jnp.concatenate([cols.astype(_I32), ar % ncols])
    vals = jnp.concatenate([vals, jnp.zeros((pad,), _F32)])
    shape = (NW, NSB, SBCH, C)
    return rows.reshape(shape), cols.reshape(shape), vals.reshape(shape)


def kernel(pois_embs, tar_rows, tar_cols, tar_vals,
           src_rows, src_cols, src_vals, layer_attention):
    tr, tc, tv = _pad_edges(tar_rows, tar_cols, tar_vals, H, N)
    sr, sc, sv = _pad_edges(src_rows, src_cols, src_vals, N, H)
    x0p = jnp.pad(pois_embs, ((0, N_PAD - N), (0, 0)))
    zeros = jnp.zeros((N_PAD, D), _F32)
    lap = jnp.concatenate(
        [layer_attention.astype(_F32),
         jnp.full((16 - layer_attention.shape[0],), -1e30, _F32)])

    t1 = _spmm_tar1(x0p, tc, tr, tv, zeros)
    s1, _ = _spmm_src(t1, sc, sr, sv, zeros)
    t2, _, r1 = _spmm_tar2(x0p, s1, tc, tr, tv, zeros)
    s2, _ = _spmm_src(t2, sc, sr, sv, zeros)
    return _final(pois_embs, r1, s2, lap)


# speedup vs baseline: 1.0869x; 1.0869x over previous
import functools

import jax
import jax.numpy as jnp
from jax import lax
from jax.experimental import pallas as pl
from jax.experimental.pallas import tpu as pltpu
from jax.experimental.pallas import tpu_sc as plsc

N, H, E, D = 10000, 5000, 320000, 128
NC, NS = 2, 16
NW = NC * NS
C = 80
NCH = 126
EPT = NCH * C
E_PAD = EPT * NW
N_PAD = 10240
H_PAD = 5120
NPT = N_PAD // NS
HPT = H_PAD // NS
NV = D // 16

_MESH = plsc.VectorSubcoreMesh(
    core_axis_name="c", subcore_axis_name="s", num_cores=NC, num_subcores=NS
)

_F32 = jnp.float32
_I32 = jnp.int32


def _row_op(n, fn):
    def body(i, carry):
        for r in range(NV):
            fn(i, pl.ds(r * 16, 16))
        return carry

    lax.fori_loop(0, n, body, 0)


SBCH = 42
NSB = NCH // SBCH


def _scale(buf, valsv, jj):
    def group(g, carry2):
        vv = valsv[jj, pl.ds(g * 16, 16)]
        for e in range(16):
            i = g * 16 + e
            v = vv[e]
            for r in range(NV):
                sl = pl.ds(r * 16, 16)
                buf[i, sl] = buf[i, sl] * v
        return carry2

    lax.fori_loop(0, C // 16, group, 0)


def _edge_pass(table_hbm, acc, wid, cols_hbm, rows_hbm, vals_hbm,
               colsv, rowsv, valsv, bufs, gsems, ssems):
    def chunk_phase(jj, b, prefetch, wait_ss):
        nb = (b + 2) % 3
        pltpu.make_async_copy(table_hbm.at[colsv.at[jj]], bufs[b],
                              gsems[b]).wait()
        _scale(bufs[b], valsv, jj)
        pltpu.async_copy(bufs[b], acc.at[rowsv.at[jj]], ssems[b], add=True)
        if prefetch:
            if wait_ss:
                pltpu.make_async_copy(bufs[nb], acc.at[rowsv.at[jj]],
                                      ssems[nb]).wait()
            pltpu.async_copy(table_hbm.at[colsv.at[jj + 2]], bufs[nb],
                             gsems[nb])

    def sb_body(sb, carry):
        pltpu.sync_copy(cols_hbm.at[wid, sb], colsv)
        pltpu.sync_copy(rows_hbm.at[wid, sb], rowsv)
        pltpu.sync_copy(vals_hbm.at[wid, sb], valsv)
        pltpu.async_copy(table_hbm.at[colsv.at[0]], bufs[0], gsems[0])
        pltpu.async_copy(table_hbm.at[colsv.at[1]], bufs[1], gsems[1])
        chunk_phase(0, 0, prefetch=True, wait_ss=False)
        chunk_phase(1, 1, prefetch=True, wait_ss=True)
        chunk_phase(2, 2, prefetch=True, wait_ss=True)

        def triple(t, carry2):
            jj = 3 * t
            chunk_phase(jj, 0, prefetch=True, wait_ss=True)
            chunk_phase(jj + 1, 1, prefetch=True, wait_ss=True)
            chunk_phase(jj + 2, 2, prefetch=True, wait_ss=True)
            return carry2

        lax.fori_loop(1, SBCH // 3 - 1, triple, 0)
        chunk_phase(SBCH - 3, 0, prefetch=True, wait_ss=True)
        chunk_phase(SBCH - 2, 1, prefetch=False, wait_ss=False)
        chunk_phase(SBCH - 1, 2, prefetch=False, wait_ss=False)
        for b in range(3):
            pltpu.make_async_copy(bufs[b], acc.at[rowsv.at[SBCH - 1]],
                                  ssems[b]).wait()
        return carry

    lax.fori_loop(0, NSB, sb_body, 0)


def _make_spmm(acc_rows):
    @functools.partial(
        pl.kernel,
        out_type=jax.ShapeDtypeStruct((NC, acc_rows, D), _F32),
        mesh=_MESH,
        scratch_types=[
            pltpu.MemorySpace.VMEM_SHARED((acc_rows, D), _F32),
            pltpu.VMEM((SBCH, C), _I32),
            pltpu.VMEM((SBCH, C), _I32),
            pltpu.VMEM((SBCH, C), _F32),
            pltpu.VMEM((C, D), _F32),
            pltpu.VMEM((C, D), _F32),
            pltpu.VMEM((C, D), _F32),
            pltpu.SemaphoreType.DMA,
            pltpu.SemaphoreType.DMA,
            pltpu.SemaphoreType.DMA,
            pltpu.SemaphoreType.DMA,
            pltpu.SemaphoreType.DMA,
            pltpu.SemaphoreType.DMA,
        ],
    )
    def spmm(x_hbm, cols_hbm, rows_hbm, vals_hbm, zeros_hbm, out_hbm,
             acc, colsv, rowsv, valsv, bufa, bufb, bufc,
             gsa, gsb, gsc, ssa, ssb, ssc):
        c = lax.axis_index("c")
        s = lax.axis_index("s")
        wid = c * NS + s
        rpt = acc_rows // NS
        pltpu.sync_copy(zeros_hbm.at[pl.ds(s * rpt, rpt)],
                        acc.at[pl.ds(s * rpt, rpt)])
        plsc.subcore_barrier()
        _edge_pass(x_hbm, acc, wid, cols_hbm, rows_hbm, vals_hbm,
                   colsv, rowsv, valsv, (bufa, bufb, bufc),
                   (gsa, gsb, gsc), (ssa, ssb, ssc))
        plsc.subcore_barrier()
        pltpu.sync_copy(acc.at[pl.ds(s * rpt, rpt)],
                        out_hbm.at[c].at[pl.ds(s * rpt, rpt)])

    return spmm


_spmm_tar = _make_spmm(H_PAD)
_spmm_src = _make_spmm(N_PAD)

SB = H_PAD // NW


@functools.partial(
    pl.kernel,
    out_type=jax.ShapeDtypeStruct((H_PAD, D), _F32),
    mesh=_MESH,
    scratch_types=[
        pltpu.VMEM((SB, D), _F32),
        pltpu.VMEM((SB, D), _F32),
        pltpu.SemaphoreType.DMA,
    ],
)
def _combine_h(p_hbm, out_hbm, cb0, cb1, sem):
    c = lax.axis_index("c")
    s = lax.axis_index("s")
    wid = c * NS + s
    off = wid * SB
    d0 = pltpu.async_copy(p_hbm.at[0].at[pl.ds(off, SB)], cb0, sem)
    d1 = pltpu.async_copy(p_hbm.at[1].at[pl.ds(off, SB)], cb1, sem)
    d0.wait()
    d1.wait()

    def combine(i, sl):
        cb0[i, sl] = cb0[i, sl] + cb1[i, sl]

    _row_op(SB, combine)
    pltpu.sync_copy(cb0, out_hbm.at[pl.ds(off, SB)])


XB = N_PAD // NW


@functools.partial(
    pl.kernel,
    out_type=(
        jax.ShapeDtypeStruct((N_PAD, D), _F32),
        jax.ShapeDtypeStruct((N_PAD, D), _F32),
    ),
    mesh=_MESH,
    scratch_types=[
        pltpu.VMEM((XB, D), _F32),
        pltpu.VMEM((XB, D), _F32),
        pltpu.VMEM((XB, D), _F32),
        pltpu.SemaphoreType.DMA,
    ],
)
def _residual(x_hbm, q_hbm, x1_hbm, r1_hbm, cbx, cb0, cb1, sem):
    c = lax.axis_index("c")
    s = lax.axis_index("s")
    wid = c * NS + s
    off = wid * XB
    d0 = pltpu.async_copy(x_hbm.at[pl.ds(off, XB)], cbx, sem)
    d1 = pltpu.async_copy(q_hbm.at[0].at[pl.ds(off, XB)], cb0, sem)
    d2 = pltpu.async_copy(q_hbm.at[1].at[pl.ds(off, XB)], cb1, sem)
    d0.wait()
    d1.wait()
    d2.wait()

    def stage(i, sl):
        r1 = jnp.maximum(cb0[i, sl] + cb1[i, sl], 0.0)
        cb0[i, sl] = r1
        cb1[i, sl] = cbx[i, sl] + r1

    _row_op(XB, stage)
    pltpu.sync_copy(cb1, x1_hbm.at[pl.ds(off, XB)])
    pltpu.sync_copy(cb0, r1_hbm.at[pl.ds(off, XB)])


KC = 80
KNCH = N // KC


@functools.partial(
    pl.kernel,
    out_type=jax.ShapeDtypeStruct((N, D), _F32),
    mesh=_MESH,
    scratch_types=[
        pltpu.VMEM((2 * KC, D), _F32),
        pltpu.VMEM((2 * KC, D), _F32),
        pltpu.VMEM((2 * KC, D), _F32),
        pltpu.VMEM((2 * KC, D), _F32),
        pltpu.VMEM((16,), _F32),
        pltpu.SemaphoreType.DMA,
        pltpu.SemaphoreType.DMA,
    ],
)
def _final(x0_hbm, r1_hbm, q_hbm, la_hbm, out_hbm, bx, b1, b2, b3, law,
           sem0, sem1):
    c = lax.axis_index("c")
    s = lax.axis_index("s")
    wid = c * NS + s
    sems = (sem0, sem1)
    pltpu.sync_copy(la_hbm, law)
    wv = law[...]
    ev = jnp.exp(wv - wv[0])
    ssum = ev[0] + ev[1] + ev[2]
    bits = lax.bitcast_convert_type(ssum, _I32)
    r = lax.bitcast_convert_type(jnp.int32(0x7EF127EA) - bits, _F32)
    for _ in range(5):
        r = r * (2.0 - ssum * r)
    w1 = ev[1] * r
    w2 = ev[2] * r
    a = w1 + w2
    nrounds = (KNCH + NW - 1) // NW

    def sources(k0):
        off = (wid + k0 * NW) * KC
        return (x0_hbm.at[pl.ds(off, KC)], r1_hbm.at[pl.ds(off, KC)],
                q_hbm.at[0].at[pl.ds(off, KC)], q_hbm.at[1].at[pl.ds(off, KC)])

    def dsts(k0):
        sl = pl.ds((k0 % 2) * KC, KC)
        return (bx.at[sl], b1.at[sl], b2.at[sl], b3.at[sl])

    def issue(k0):
        sem = sems[k0 % 2]
        for src, dst in zip(sources(k0), dsts(k0)):
            pltpu.async_copy(src, dst, sem)

    issue(0)
    for k0 in range(nrounds):
        j = wid + k0 * NW

        @pl.when(j < KNCH)
        def _(k0=k0, j=j):
            sem = sems[k0 % 2]
            for src, dst in zip(sources(k0), dsts(k0)):
                pltpu.make_async_copy(src, dst, sem).wait()
            if k0 + 1 < nrounds:

                @pl.when(wid + (k0 + 1) * NW < KNCH)
                def _():
                    issue(k0 + 1)

            base = (k0 % 2) * KC

            def mix(i, sl):
                ii = base + i
                r2 = jnp.maximum(b2[ii, sl] + b3[ii, sl], 0.0)
                bx[ii, sl] = bx[ii, sl] + a * b1[ii, sl] + w2 * r2

            _row_op(KC, mix)
            pltpu.sync_copy(bx.at[pl.ds(base, KC)],
                            out_hbm.at[pl.ds(j * KC, KC)])


def _pad_edges(rows, cols, vals, nrows, ncols):
    pad = E_PAD - E
    ar = jnp.arange(pad, dtype=_I32)
    rows = jnp.concatenate([rows.astype(_I32), ar % nrows])
    cols = jnp.concatenate([cols.astype(_I32), ar % ncols])
    vals = jnp.concatenate([vals, jnp.zeros((pad,), _F32)])
    shape = (NW, NSB, SBCH, C)
    return rows.reshape(shape), cols.reshape(shape), vals.reshape(shape)


def kernel(pois_embs, tar_rows, tar_cols, tar_vals,
           src_rows, src_cols, src_vals, layer_attention):
    tr, tc, tv = _pad_edges(tar_rows, tar_cols, tar_vals, H, N)
    sr, sc, sv = _pad_edges(src_rows, src_cols, src_vals, N, H)
    x0p = jnp.pad(pois_embs, ((0, N_PAD - N), (0, 0)))
    zeros = jnp.zeros((N_PAD, D), _F32)
    lap = jnp.concatenate(
        [layer_attention.astype(_F32),
         jnp.full((16 - layer_attention.shape[0],), -1e30, _F32)])

    t1 = _spmm_tar(pois_embs, tc, tr, tv, zeros)
    mt1 = _combine_h(t1)
    s1 = _spmm_src(mt1, sc, sr, sv, zeros)
    x1, r1 = _residual(x0p, s1)
    t2 = _spmm_tar(x1, tc, tr, tv, zeros)
    mt2 = _combine_h(t2)
    s2 = _spmm_src(mt2, sc, sr, sv, zeros)
    return _final(pois_embs, r1, s2, lap)


# speedup vs baseline: 1.1204x; 1.0308x over previous
import functools

import jax
import jax.numpy as jnp
from jax import lax
from jax.experimental import pallas as pl
from jax.experimental.pallas import tpu as pltpu
from jax.experimental.pallas import tpu_sc as plsc

N, H, E, D = 10000, 5000, 320000, 128
NC, NS = 2, 16
NW = NC * NS
C = 80
NCH = 126
EPT = NCH * C
E_PAD = EPT * NW
N_PAD = 10240
H_PAD = 5120
NPT = N_PAD // NS
HPT = H_PAD // NS
NV = D // 16

_MESH = plsc.VectorSubcoreMesh(
    core_axis_name="c", subcore_axis_name="s", num_cores=NC, num_subcores=NS
)

_F32 = jnp.float32
_I32 = jnp.int32


def _row_op(n, fn):
    def body(i, carry):
        for r in range(NV):
            fn(i, pl.ds(r * 16, 16))
        return carry

    lax.fori_loop(0, n, body, 0)


SBCH = 42
NSB = NCH // SBCH


def _scale(buf, valsv, jj):
    def group(g, carry2):
        vv = valsv[jj, pl.ds(g * 16, 16)]
        for e in range(16):
            i = g * 16 + e
            v = vv[e]
            for r in range(NV):
                sl = pl.ds(r * 16, 16)
                buf[i, sl] = buf[i, sl] * v
        return carry2

    lax.fori_loop(0, C // 16, group, 0)


def _edge_pass(table_hbm, acc, wid, cols_hbm, rows_hbm, vals_hbm,
               colsv, rowsv, valsv, bufs, gsems, ssems, isem):
    def chunk_phase(jj, b, prefetch, wait_ss):
        nb = (b + 2) % 3
        pltpu.make_async_copy(table_hbm.at[colsv.at[jj]], bufs[b],
                              gsems[b]).wait()
        _scale(bufs[b], valsv, jj)
        pltpu.async_copy(bufs[b], acc.at[rowsv.at[jj]], ssems[b], add=True)
        if prefetch:
            if wait_ss:
                pltpu.make_async_copy(bufs[nb], acc.at[rowsv.at[jj]],
                                      ssems[nb]).wait()
            pltpu.async_copy(table_hbm.at[colsv.at[jj + 2]], bufs[nb],
                             gsems[nb])

    def sb_body(sb, carry):
        d0 = pltpu.async_copy(cols_hbm.at[wid, sb], colsv, isem)
        d1 = pltpu.async_copy(rows_hbm.at[wid, sb], rowsv, isem)
        d2 = pltpu.async_copy(vals_hbm.at[wid, sb], valsv, isem)
        d0.wait()
        d1.wait()
        d2.wait()
        pltpu.async_copy(table_hbm.at[colsv.at[0]], bufs[0], gsems[0])
        pltpu.async_copy(table_hbm.at[colsv.at[1]], bufs[1], gsems[1])
        chunk_phase(0, 0, prefetch=True, wait_ss=False)
        chunk_phase(1, 1, prefetch=True, wait_ss=True)
        chunk_phase(2, 2, prefetch=True, wait_ss=True)

        def triple(t, carry2):
            jj = 3 * t
            chunk_phase(jj, 0, prefetch=True, wait_ss=True)
            chunk_phase(jj + 1, 1, prefetch=True, wait_ss=True)
            chunk_phase(jj + 2, 2, prefetch=True, wait_ss=True)
            return carry2

        lax.fori_loop(1, SBCH // 3 - 1, triple, 0)
        chunk_phase(SBCH - 3, 0, prefetch=True, wait_ss=True)
        chunk_phase(SBCH - 2, 1, prefetch=False, wait_ss=False)
        chunk_phase(SBCH - 1, 2, prefetch=False, wait_ss=False)
        for b in range(3):
            pltpu.make_async_copy(bufs[b], acc.at[rowsv.at[SBCH - 1]],
                                  ssems[b]).wait()
        return carry

    lax.fori_loop(0, NSB, sb_body, 0)


def _make_spmm(acc_rows):
    @functools.partial(
        pl.kernel,
        out_type=jax.ShapeDtypeStruct((NC, acc_rows, D), _F32),
        mesh=_MESH,
        scratch_types=[
            pltpu.MemorySpace.VMEM_SHARED((acc_rows, D), _F32),
            pltpu.VMEM((SBCH, C), _I32),
            pltpu.VMEM((SBCH, C), _I32),
            pltpu.VMEM((SBCH, C), _F32),
            pltpu.VMEM((C, D), _F32),
            pltpu.VMEM((C, D), _F32),
            pltpu.VMEM((C, D), _F32),
            pltpu.SemaphoreType.DMA,
            pltpu.SemaphoreType.DMA,
            pltpu.SemaphoreType.DMA,
            pltpu.SemaphoreType.DMA,
            pltpu.SemaphoreType.DMA,
            pltpu.SemaphoreType.DMA,
            pltpu.SemaphoreType.DMA,
        ],
    )
    def spmm(x_hbm, cols_hbm, rows_hbm, vals_hbm, zeros_hbm, out_hbm,
             acc, colsv, rowsv, valsv, bufa, bufb, bufc,
             gsa, gsb, gsc, ssa, ssb, ssc, isem):
        c = lax.axis_index("c")
        s = lax.axis_index("s")
        wid = c * NS + s
        rpt = acc_rows // NS
        pltpu.sync_copy(zeros_hbm.at[pl.ds(s * rpt, rpt)],
                        acc.at[pl.ds(s * rpt, rpt)])
        plsc.subcore_barrier()
        _edge_pass(x_hbm, acc, wid, cols_hbm, rows_hbm, vals_hbm,
                   colsv, rowsv, valsv, (bufa, bufb, bufc),
                   (gsa, gsb, gsc), (ssa, ssb, ssc), isem)
        plsc.subcore_barrier()
        pltpu.sync_copy(acc.at[pl.ds(s * rpt, rpt)],
                        out_hbm.at[c].at[pl.ds(s * rpt, rpt)])

    return spmm


_spmm_tar = _make_spmm(H_PAD)
_spmm_src = _make_spmm(N_PAD)

SB = H_PAD // NW


@functools.partial(
    pl.kernel,
    out_type=jax.ShapeDtypeStruct((H_PAD, D), _F32),
    mesh=_MESH,
    scratch_types=[
        pltpu.VMEM((SB, D), _F32),
        pltpu.VMEM((SB, D), _F32),
        pltpu.SemaphoreType.DMA,
    ],
)
def _combine_h(p_hbm, out_hbm, cb0, cb1, sem):
    c = lax.axis_index("c")
    s = lax.axis_index("s")
    wid = c * NS + s
    off = wid * SB
    d0 = pltpu.async_copy(p_hbm.at[0].at[pl.ds(off, SB)], cb0, sem)
    d1 = pltpu.async_copy(p_hbm.at[1].at[pl.ds(off, SB)], cb1, sem)
    d0.wait()
    d1.wait()

    def combine(i, sl):
        cb0[i, sl] = cb0[i, sl] + cb1[i, sl]

    _row_op(SB, combine)
    pltpu.sync_copy(cb0, out_hbm.at[pl.ds(off, SB)])


XB = N_PAD // NW


@functools.partial(
    pl.kernel,
    out_type=(
        jax.ShapeDtypeStruct((N_PAD, D), _F32),
        jax.ShapeDtypeStruct((N_PAD, D), _F32),
    ),
    mesh=_MESH,
    scratch_types=[
        pltpu.VMEM((XB, D), _F32),
        pltpu.VMEM((XB, D), _F32),
        pltpu.VMEM((XB, D), _F32),
        pltpu.SemaphoreType.DMA,
    ],
)
def _residual(x_hbm, q_hbm, x1_hbm, r1_hbm, cbx, cb0, cb1, sem):
    c = lax.axis_index("c")
    s = lax.axis_index("s")
    wid = c * NS + s
    off = wid * XB
    d0 = pltpu.async_copy(x_hbm.at[pl.ds(off, XB)], cbx, sem)
    d1 = pltpu.async_copy(q_hbm.at[0].at[pl.ds(off, XB)], cb0, sem)
    d2 = pltpu.async_copy(q_hbm.at[1].at[pl.ds(off, XB)], cb1, sem)
    d0.wait()
    d1.wait()
    d2.wait()

    def stage(i, sl):
        r1 = jnp.maximum(cb0[i, sl] + cb1[i, sl], 0.0)
        cb0[i, sl] = r1
        cb1[i, sl] = cbx[i, sl] + r1

    _row_op(XB, stage)
    pltpu.sync_copy(cb1, x1_hbm.at[pl.ds(off, XB)])
    pltpu.sync_copy(cb0, r1_hbm.at[pl.ds(off, XB)])


KC = 80
KNCH = N // KC


@functools.partial(
    pl.kernel,
    out_type=jax.ShapeDtypeStruct((N, D), _F32),
    mesh=_MESH,
    scratch_types=[
        pltpu.VMEM((2 * KC, D), _F32),
        pltpu.VMEM((2 * KC, D), _F32),
        pltpu.VMEM((2 * KC, D), _F32),
        pltpu.VMEM((2 * KC, D), _F32),
        pltpu.VMEM((16,), _F32),
        pltpu.SemaphoreType.DMA,
        pltpu.SemaphoreType.DMA,
    ],
)
def _final(x0_hbm, r1_hbm, q_hbm, la_hbm, out_hbm, bx, b1, b2, b3, law,
           sem0, sem1):
    c = lax.axis_index("c")
    s = lax.axis_index("s")
    wid = c * NS + s
    sems = (sem0, sem1)
    pltpu.sync_copy(la_hbm, law)
    wv = law[...]
    ev = jnp.exp(wv - wv[0])
    ssum = ev[0] + ev[1] + ev[2]
    bits = lax.bitcast_convert_type(ssum, _I32)
    r = lax.bitcast_convert_type(jnp.int32(0x7EF127EA) - bits, _F32)
    for _ in range(5):
        r = r * (2.0 - ssum * r)
    w1 = ev[1] * r
    w2 = ev[2] * r
    a = w1 + w2
    nrounds = (KNCH + NW - 1) // NW

    def sources(k0):
        off = (wid + k0 * NW) * KC
        return (x0_hbm.at[pl.ds(off, KC)], r1_hbm.at[pl.ds(off, KC)],
                q_hbm.at[0].at[pl.ds(off, KC)], q_hbm.at[1].at[pl.ds(off, KC)])

    def dsts(k0):
        sl = pl.ds((k0 % 2) * KC, KC)
        return (bx.at[sl], b1.at[sl], b2.at[sl], b3.at[sl])

    def issue(k0):
        sem = sems[k0 % 2]
        for src, dst in zip(sources(k0), dsts(k0)):
            pltpu.async_copy(src, dst, sem)

    issue(0)
    for k0 in range(nrounds):
        j = wid + k0 * NW

        @pl.when(j < KNCH)
        def _(k0=k0, j=j):
            sem = sems[k0 % 2]
            for src, dst in zip(sources(k0), dsts(k0)):
                pltpu.make_async_copy(src, dst, sem).wait()
            if k0 + 1 < nrounds:

                @pl.when(wid + (k0 + 1) * NW < KNCH)
                def _():
                    issue(k0 + 1)

            base = (k0 % 2) * KC

            def mix(i, sl):
                ii = base + i
                r2 = jnp.maximum(b2[ii, sl] + b3[ii, sl], 0.0)
                bx[ii, sl] = bx[ii, sl] + a * b1[ii, sl] + w2 * r2

            _row_op(KC, mix)
            pltpu.sync_copy(bx.at[pl.ds(base, KC)],
                            out_hbm.at[pl.ds(j * KC, KC)])


def _pad_edges(rows, cols, vals, nrows, ncols):
    pad = E_PAD - E
    ar = jnp.arange(pad, dtype=_I32)
    rows = jnp.concatenate([rows.astype(_I32), ar % nrows])
    cols = jnp.concatenate([cols.astype(_I32), ar % ncols])
    vals = jnp.concatenate([vals, jnp.zeros((pad,), _F32)])
    shape = (NW, NSB, SBCH, C)
    return rows.reshape(shape), cols.reshape(shape), vals.reshape(shape)


def kernel(pois_embs, tar_rows, tar_cols, tar_vals,
           src_rows, src_cols, src_vals, layer_attention):
    tr, tc, tv = _pad_edges(tar_rows, tar_cols, tar_vals, H, N)
    sr, sc, sv = _pad_edges(src_rows, src_cols, src_vals, N, H)
    x0p = jnp.pad(pois_embs, ((0, N_PAD - N), (0, 0)))
    zeros = jnp.zeros((N_PAD, D), _F32)
    lap = jnp.concatenate(
        [layer_attention.astype(_F32),
         jnp.full((16 - layer_attention.shape[0],), -1e30, _F32)])

    t1 = _spmm_tar(pois_embs, tc, tr, tv, zeros)
    mt1 = _combine_h(t1)
    s1 = _spmm_src(mt1, sc, sr, sv, zeros)
    x1, r1 = _residual(x0p, s1)
    t2 = _spmm_tar(x1, tc, tr, tv, zeros)
    mt2 = _combine_h(t2)
    s2 = _spmm_src(mt2, sc, sr, sv, zeros)
    return _final(pois_embs, r1, s2, lap)


# speedup vs baseline: 1.1261x; 1.0051x over previous
import functools

import jax
import jax.numpy as jnp
from jax import lax
from jax.experimental import pallas as pl
from jax.experimental.pallas import tpu as pltpu
from jax.experimental.pallas import tpu_sc as plsc

N, H, E, D = 10000, 5000, 320000, 128
NC, NS = 2, 16
NW = NC * NS
C = 80
NCH = 126
EPT = NCH * C
E_PAD = EPT * NW
N_PAD = 10240
H_PAD = 5120
NPT = N_PAD // NS
HPT = H_PAD // NS
NV = D // 16

_MESH = plsc.VectorSubcoreMesh(
    core_axis_name="c", subcore_axis_name="s", num_cores=NC, num_subcores=NS
)

_F32 = jnp.float32
_I32 = jnp.int32


def _row_op(n, fn):
    def body(i, carry):
        for r in range(NV):
            fn(i, pl.ds(r * 16, 16))
        return carry

    lax.fori_loop(0, n, body, 0)


SBCH = 42
NSB = NCH // SBCH


def _scale(buf, valsv, jj, g0, g1):
    def group(g, carry2):
        vv = valsv[jj, pl.ds(g * 16, 16)]
        for e in range(16):
            i = g * 16 + e
            v = vv[e]
            for r in range(NV):
                sl = pl.ds(r * 16, 16)
                buf[i, sl] = buf[i, sl] * v
        return carry2

    lax.fori_loop(g0, g1, group, 0)


def _edge_pass(table_hbm, acc, wid, cols_hbm, rows_hbm, vals_hbm,
               colsv, rowsv, valsv, bufs, gsems, ssems, isem):
    def chunk_phase(jj, b, prefetch, wait_ss):
        nb = (b + 2) % 3
        pltpu.make_async_copy(table_hbm.at[colsv.at[jj]], bufs[b],
                              gsems[b]).wait()
        _scale(bufs[b], valsv, jj, 0, 2)
        if prefetch:
            if wait_ss:
                pltpu.make_async_copy(bufs[nb], acc.at[rowsv.at[jj]],
                                      ssems[nb]).wait()
            pltpu.async_copy(table_hbm.at[colsv.at[jj + 2]], bufs[nb],
                             gsems[nb])
        _scale(bufs[b], valsv, jj, 2, C // 16)
        pltpu.async_copy(bufs[b], acc.at[rowsv.at[jj]], ssems[b], add=True)

    def sb_body(sb, carry):
        d0 = pltpu.async_copy(cols_hbm.at[wid, sb], colsv, isem)
        d1 = pltpu.async_copy(rows_hbm.at[wid, sb], rowsv, isem)
        d2 = pltpu.async_copy(vals_hbm.at[wid, sb], valsv, isem)
        d0.wait()
        d1.wait()
        d2.wait()
        pltpu.async_copy(table_hbm.at[colsv.at[0]], bufs[0], gsems[0])
        pltpu.async_copy(table_hbm.at[colsv.at[1]], bufs[1], gsems[1])
        chunk_phase(0, 0, prefetch=True, wait_ss=False)
        chunk_phase(1, 1, prefetch=True, wait_ss=True)
        chunk_phase(2, 2, prefetch=True, wait_ss=True)

        def triple(t, carry2):
            jj = 3 * t
            chunk_phase(jj, 0, prefetch=True, wait_ss=True)
            chunk_phase(jj + 1, 1, prefetch=True, wait_ss=True)
            chunk_phase(jj + 2, 2, prefetch=True, wait_ss=True)
            return carry2

        lax.fori_loop(1, SBCH // 3 - 1, triple, 0)
        chunk_phase(SBCH - 3, 0, prefetch=True, wait_ss=True)
        chunk_phase(SBCH - 2, 1, prefetch=False, wait_ss=False)
        chunk_phase(SBCH - 1, 2, prefetch=False, wait_ss=False)
        for b in range(3):
            pltpu.make_async_copy(bufs[b], acc.at[rowsv.at[SBCH - 1]],
                                  ssems[b]).wait()
        return carry

    lax.fori_loop(0, NSB, sb_body, 0)


def _make_spmm(acc_rows):
    @functools.partial(
        pl.kernel,
        out_type=jax.ShapeDtypeStruct((NC, acc_rows, D), _F32),
        mesh=_MESH,
        scratch_types=[
            pltpu.MemorySpace.VMEM_SHARED((acc_rows, D), _F32),
            pltpu.VMEM((SBCH, C), _I32),
            pltpu.VMEM((SBCH, C), _I32),
            pltpu.VMEM((SBCH, C), _F32),
            pltpu.VMEM((C, D), _F32),
            pltpu.VMEM((C, D), _F32),
            pltpu.VMEM((C, D), _F32),
            pltpu.SemaphoreType.DMA,
            pltpu.SemaphoreType.DMA,
            pltpu.SemaphoreType.DMA,
            pltpu.SemaphoreType.DMA,
            pltpu.SemaphoreType.DMA,
            pltpu.SemaphoreType.DMA,
            pltpu.SemaphoreType.DMA,
        ],
    )
    def spmm(x_hbm, cols_hbm, rows_hbm, vals_hbm, zeros_hbm, out_hbm,
             acc, colsv, rowsv, valsv, bufa, bufb, bufc,
             gsa, gsb, gsc, ssa, ssb, ssc, isem):
        c = lax.axis_index("c")
        s = lax.axis_index("s")
        wid = c * NS + s
        rpt = acc_rows // NS
        pltpu.sync_copy(zeros_hbm.at[pl.ds(s * rpt, rpt)],
                        acc.at[pl.ds(s * rpt, rpt)])
        plsc.subcore_barrier()
        _edge_pass(x_hbm, acc, wid, cols_hbm, rows_hbm, vals_hbm,
                   colsv, rowsv, valsv, (bufa, bufb, bufc),
                   (gsa, gsb, gsc), (ssa, ssb, ssc), isem)
        plsc.subcore_barrier()
        pltpu.sync_copy(acc.at[pl.ds(s * rpt, rpt)],
                        out_hbm.at[c].at[pl.ds(s * rpt, rpt)])

    return spmm


_spmm_tar = _make_spmm(H_PAD)
_spmm_src = _make_spmm(N_PAD)

SB = H_PAD // NW


@functools.partial(
    pl.kernel,
    out_type=jax.ShapeDtypeStruct((H_PAD, D), _F32),
    mesh=_MESH,
    scratch_types=[
        pltpu.VMEM((SB, D), _F32),
        pltpu.VMEM((SB, D), _F32),
        pltpu.SemaphoreType.DMA,
    ],
)
def _combine_h(p_hbm, out_hbm, cb0, cb1, sem):
    c = lax.axis_index("c")
    s = lax.axis_index("s")
    wid = c * NS + s
    off = wid * SB
    d0 = pltpu.async_copy(p_hbm.at[0].at[pl.ds(off, SB)], cb0, sem)
    d1 = pltpu.async_copy(p_hbm.at[1].at[pl.ds(off, SB)], cb1, sem)
    d0.wait()
    d1.wait()

    def combine(i, sl):
        cb0[i, sl] = cb0[i, sl] + cb1[i, sl]

    _row_op(SB, combine)
    pltpu.sync_copy(cb0, out_hbm.at[pl.ds(off, SB)])


XB = N_PAD // NW


@functools.partial(
    pl.kernel,
    out_type=(
        jax.ShapeDtypeStruct((N_PAD, D), _F32),
        jax.ShapeDtypeStruct((N_PAD, D), _F32),
    ),
    mesh=_MESH,
    scratch_types=[
        pltpu.VMEM((XB, D), _F32),
        pltpu.VMEM((XB, D), _F32),
        pltpu.VMEM((XB, D), _F32),
        pltpu.SemaphoreType.DMA,
    ],
)
def _residual(x_hbm, q_hbm, x1_hbm, r1_hbm, cbx, cb0, cb1, sem):
    c = lax.axis_index("c")
    s = lax.axis_index("s")
    wid = c * NS + s
    off = wid * XB
    d0 = pltpu.async_copy(x_hbm.at[pl.ds(off, XB)], cbx, sem)
    d1 = pltpu.async_copy(q_hbm.at[0].at[pl.ds(off, XB)], cb0, sem)
    d2 = pltpu.async_copy(q_hbm.at[1].at[pl.ds(off, XB)], cb1, sem)
    d0.wait()
    d1.wait()
    d2.wait()

    def stage(i, sl):
        r1 = jnp.maximum(cb0[i, sl] + cb1[i, sl], 0.0)
        cb0[i, sl] = r1
        cb1[i, sl] = cbx[i, sl] + r1

    _row_op(XB, stage)
    pltpu.sync_copy(cb1, x1_hbm.at[pl.ds(off, XB)])
    pltpu.sync_copy(cb0, r1_hbm.at[pl.ds(off, XB)])


KC = 80
KNCH = N // KC


@functools.partial(
    pl.kernel,
    out_type=jax.ShapeDtypeStruct((N, D), _F32),
    mesh=_MESH,
    scratch_types=[
        pltpu.VMEM((2 * KC, D), _F32),
        pltpu.VMEM((2 * KC, D), _F32),
        pltpu.VMEM((2 * KC, D), _F32),
        pltpu.VMEM((2 * KC, D), _F32),
        pltpu.VMEM((16,), _F32),
        pltpu.SemaphoreType.DMA,
        pltpu.SemaphoreType.DMA,
    ],
)
def _final(x0_hbm, r1_hbm, q_hbm, la_hbm, out_hbm, bx, b1, b2, b3, law,
           sem0, sem1):
    c = lax.axis_index("c")
    s = lax.axis_index("s")
    wid = c * NS + s
    sems = (sem0, sem1)
    pltpu.sync_copy(la_hbm, law)
    wv = law[...]
    ev = jnp.exp(wv - wv[0])
    ssum = ev[0] + ev[1] + ev[2]
    bits = lax.bitcast_convert_type(ssum, _I32)
    r = lax.bitcast_convert_type(jnp.int32(0x7EF127EA) - bits, _F32)
    for _ in range(5):
        r = r * (2.0 - ssum * r)
    w1 = ev[1] * r
    w2 = ev[2] * r
    a = w1 + w2
    nrounds = (KNCH + NW - 1) // NW

    def sources(k0):
        off = (wid + k0 * NW) * KC
        return (x0_hbm.at[pl.ds(off, KC)], r1_hbm.at[pl.ds(off, KC)],
                q_hbm.at[0].at[pl.ds(off, KC)], q_hbm.at[1].at[pl.ds(off, KC)])

    def dsts(k0):
        sl = pl.ds((k0 % 2) * KC, KC)
        return (bx.at[sl], b1.at[sl], b2.at[sl], b3.at[sl])

    def issue(k0):
        sem = sems[k0 % 2]
        for src, dst in zip(sources(k0), dsts(k0)):
            pltpu.async_copy(src, dst, sem)

    issue(0)
    for k0 in range(nrounds):
        j = wid + k0 * NW

        @pl.when(j < KNCH)
        def _(k0=k0, j=j):
            sem = sems[k0 % 2]
            for src, dst in zip(sources(k0), dsts(k0)):
                pltpu.make_async_copy(src, dst, sem).wait()
            if k0 + 1 < nrounds:

                @pl.when(wid + (k0 + 1) * NW < KNCH)
                def _():
                    issue(k0 + 1)

            base = (k0 % 2) * KC

            def mix(i, sl):
                ii = base + i
                r2 = jnp.maximum(b2[ii, sl] + b3[ii, sl], 0.0)
                bx[ii, sl] = bx[ii, sl] + a * b1[ii, sl] + w2 * r2

            _row_op(KC, mix)
            pltpu.sync_copy(bx.at[pl.ds(base, KC)],
                            out_hbm.at[pl.ds(j * KC, KC)])


def _pad_edges(rows, cols, vals, nrows, ncols):
    pad = E_PAD - E
    ar = jnp.arange(pad, dtype=_I32)
    rows = jnp.concatenate([rows.astype(_I32), ar % nrows])
    cols = jnp.concatenate([cols.astype(_I32), ar % ncols])
    vals = jnp.concatenate([vals, jnp.zeros((pad,), _F32)])
    shape = (NW, NSB, SBCH, C)
    return rows.reshape(shape), cols.reshape(shape), vals.reshape(shape)


def kernel(pois_embs, tar_rows, tar_cols, tar_vals,
           src_rows, src_cols, src_vals, layer_attention):
    tr, tc, tv = _pad_edges(tar_rows, tar_cols, tar_vals, H, N)
    sr, sc, sv = _pad_edges(src_rows, src_cols, src_vals, N, H)
    x0p = jnp.pad(pois_embs, ((0, N_PAD - N), (0, 0)))
    zeros = jnp.zeros((N_PAD, D), _F32)
    lap = jnp.concatenate(
        [layer_attention.astype(_F32),
         jnp.full((16 - layer_attention.shape[0],), -1e30, _F32)])

    t1 = _spmm_tar(pois_embs, tc, tr, tv, zeros)
    mt1 = _combine_h(t1)
    s1 = _spmm_src(mt1, sc, sr, sv, zeros)
    x1, r1 = _residual(x0p, s1)
    t2 = _spmm_tar(x1, tc, tr, tv, zeros)
    mt2 = _combine_h(t2)
    s2 = _spmm_src(mt2, sc, sr, sv, zeros)
    return _final(pois_embs, r1, s2, lap)
